# baseline (speedup 1.0000x reference)
import functools

import jax
import jax.numpy as jnp
from jax import lax
from jax.experimental import pallas as pl
from jax.experimental.pallas import tpu as pltpu
from jax.experimental.pallas import tpu_sc as plsc

E = 8
K = 2
T = 2048
D = 768
P = T * K
MM_TILE = 128
NS = P + E * MM_TILE
NT = NS // MM_TILE
NTP = 48
NC = 2
NSUB = 16
TPS = T // NSUB
PPS = TPS * K
SPW = NS // (NC * NSUB)
NEG_INF = float("-inf")


def _iota16():
    return lax.broadcasted_iota(jnp.int32, (16,), 0)


def _lane_gather(src, idx):
    return lax.gather(
        src, idx[:, None],
        lax.GatherDimensionNumbers(
            offset_dims=(), collapsed_slice_dims=(0,), start_index_map=(0,)),
        slice_sizes=(1,),
        mode=lax.GatherScatterMode.PROMISE_IN_BOUNDS)


def _lane_sum(x):
    iot = _iota16()
    for d in (1, 2, 4, 8):
        x = x + _lane_gather(x, iot ^ d)
    return x


def _lane_prefix(x):
    iot = _iota16()
    for d in (1, 2, 4, 8):
        sh = _lane_gather(x, jnp.maximum(iot - d, 0))
        x = x + jnp.where(iot >= d, sh, 0)
    return x


def _lane_bcast(x, e):
    return _lane_gather(x, jnp.full((16,), e, jnp.int32))


def _dispatch_body(logits_ref, hidden_ref,
                   xs_ref, ws_ref, pos_ref, texp_ref,
                   lg, epair, wpair, histv, allh, texp_v, slots, toks,
                   idx2, rows, ws_v, shared_hist, shared_src, shared_ws, sem):
    c = lax.axis_index("c")
    s = lax.axis_index("s")

    pltpu.sync_copy(logits_ref.at[:, pl.ds(TPS * s, TPS)], lg)
    for g in range(TPS // 16):
        l_vecs = [lg[e, pl.ds(16 * g, 16)] for e in range(E)]
        best = l_vecs[0]
        bi = jnp.zeros((16,), jnp.int32)
        for e in range(1, E):
            m = l_vecs[e] > best
            best = jnp.where(m, l_vecs[e], best)
            bi = jnp.where(m, e, bi)
        sec = jnp.full((16,), NEG_INF, jnp.float32)
        si = jnp.zeros((16,), jnp.int32)
        for e in range(E):
            m = jnp.logical_and(bi != e, l_vecs[e] > sec)
            sec = jnp.where(m, l_vecs[e], sec)
            si = jnp.where(m, e, si)
        e2 = jnp.exp(sec - best)
        w1 = 1.0 / (1.0 + e2)
        epair[pl.ds(16 * g, 16)] = bi
        epair[pl.ds(TPS + 16 * g, 16)] = si
        wpair[pl.ds(16 * g, 16)] = w1
        wpair[pl.ds(TPS + 16 * g, 16)] = e2 * w1

    iot = _iota16()
    hist = jnp.zeros((16,), jnp.int32)
    for ch in range(PPS // 16):
        ev = epair[pl.ds(16 * ch, 16)]
        for e in range(E):
            pc = _lane_sum(jnp.where(ev == e, 1, 0))
            hist = hist + jnp.where(iot == e, pc, 0)
    histv[...] = hist
    pltpu.sync_copy(histv, shared_hist.at[s])
    plsc.subcore_barrier()

    pltpu.sync_copy(shared_hist, allh)
    tot = jnp.zeros((16,), jnp.int32)
    pre = jnp.zeros((16,), jnp.int32)
    for w in range(NSUB):
        row = allh[w]
        tot = tot + row
        gate = (jnp.int32(w) < s).astype(jnp.int32)
        pre = pre + row * gate
    padded = ((tot + (MM_TILE - 1)) >> 7) << 7
    incl = _lane_prefix(padded)
    base = (incl - padded) + pre

    @pl.when(jnp.logical_and(c == 0, s == 0))
    def _texp():
        ies = [_lane_bcast(incl, e) for e in range(E)]
        for vi in range(NTP // 16):
            startv = (iot + 16 * vi) * MM_TILE
            below = jnp.zeros((16,), jnp.int32)
            for e in range(E):
                below = below + jnp.where(ies[e] <= startv, 1, 0)
            texp_v[pl.ds(16 * vi, 16)] = jnp.minimum(below, E - 1)
        pltpu.sync_copy(texp_v, texp_ref)

    run = base
    for ch in range(PPS // 16):
        ev = epair[pl.ds(16 * ch, 16)]
        sv = jnp.zeros((16,), jnp.int32)
        for e in range(E):
            m = ev == e
            r = _lane_prefix(jnp.where(m, 1, 0))
            cnt = _lane_bcast(r, 15)
            be = _lane_bcast(run, e)
            sv = jnp.where(m, be + (r - 1), sv)
            run = run + jnp.where(iot == e, cnt, 0)
        slots[pl.ds(16 * ch, 16)] = sv
        toks[pl.ds(16 * ch, 16)] = ((iot + 16 * ch) & (TPS - 1)) + TPS * s

    @pl.when(c == 0)
    def _pos():
        pltpu.sync_copy(slots.at[pl.ds(0, TPS)], pos_ref.at[0, pl.ds(TPS * s, TPS)])
        pltpu.sync_copy(slots.at[pl.ds(TPS, TPS)], pos_ref.at[1, pl.ds(TPS * s, TPS)])

    pltpu.sync_copy(toks, shared_src.at[slots])
    pltpu.sync_copy(wpair, shared_ws.at[slots])
    plsc.subcore_barrier()

    start = SPW * (NSUB * c + s)
    for h in range(2):
        hw = SPW // 2
        pltpu.sync_copy(shared_src.at[pl.ds(start + hw * h, hw)], idx2.at[h])
        for j in range(hw // 16):
            v = idx2[h, pl.ds(16 * j, 16)]
            idx2[h, pl.ds(16 * j, 16)] = jnp.clip(v, 0, T - 1)
        pltpu.async_copy(hidden_ref.at[idx2.at[h]],
                         rows.at[pl.ds(hw * h, hw)], sem).wait()
    pltpu.sync_copy(rows, xs_ref.at[pl.ds(start, SPW)])
    pltpu.sync_copy(shared_ws.at[pl.ds(start, SPW)], ws_v)
    pltpu.sync_copy(ws_v, ws_ref.at[pl.ds(start, SPW)])


_dispatch = functools.partial(
    pl.kernel,
    out_type=[
        jax.ShapeDtypeStruct((NS, D), jnp.float32),
        jax.ShapeDtypeStruct((NS,), jnp.float32),
        jax.ShapeDtypeStruct((K, T), jnp.int32),
        jax.ShapeDtypeStruct((NTP,), jnp.int32),
    ],
    mesh=plsc.VectorSubcoreMesh(core_axis_name="c", subcore_axis_name="s"),
    scratch_types=[
        pltpu.VMEM((E, TPS), jnp.float32),
        pltpu.VMEM((PPS,), jnp.int32),
        pltpu.VMEM((PPS,), jnp.float32),
        pltpu.VMEM((16,), jnp.int32),
        pltpu.VMEM((NSUB, 16), jnp.int32),
        pltpu.VMEM((NTP,), jnp.int32),
        pltpu.VMEM((PPS,), jnp.int32),
        pltpu.VMEM((PPS,), jnp.int32),
        pltpu.VMEM((2, SPW // 2), jnp.int32),
        pltpu.VMEM((SPW, D), jnp.float32),
        pltpu.VMEM((SPW,), jnp.float32),
        pltpu.VMEM_SHARED((NSUB, 16), jnp.int32),
        pltpu.VMEM_SHARED((NS,), jnp.int32),
        pltpu.VMEM_SHARED((NS,), jnp.float32),
        pltpu.SemaphoreType.DMA,
    ],
)(_dispatch_body)


def _mm_body(texp_ref, x_ref, w_ref, b_ref, ws_ref, y_ref):
    xb = x_ref[...].astype(jnp.bfloat16)
    wb = w_ref[0].astype(jnp.bfloat16)
    y = jax.lax.dot_general(
        xb, wb, (((1,), (0,)), ((), ())),
        preferred_element_type=jnp.float32,
    ) + b_ref[0]
    y_ref[...] = y * ws_ref[0, 0][:, None]


def _combine_body(y_ref, pos_ref, out_ref, p0, p1, y0, y1, sem0, sem1):
    wid = lax.axis_index("s") * NC + lax.axis_index("c")
    tw = T // (NC * NSUB)
    base = tw * wid
    pltpu.sync_copy(pos_ref.at[0, pl.ds(base, tw)], p0)
    pltpu.sync_copy(pos_ref.at[1, pl.ds(base, tw)], p1)
    cp0 = pltpu.async_copy(y_ref.at[p0], y0, sem0)
    cp1 = pltpu.async_copy(y_ref.at[p1], y1, sem1)
    cp0.wait()
    cp1.wait()

    def body(t, carry):
        for v in range(D // 16):
            sl = pl.ds(16 * v, 16)
            y0[t, sl] = y0[t, sl] + y1[t, sl]
        return carry

    lax.fori_loop(0, tw, body, 0)
    pltpu.sync_copy(y0, out_ref.at[pl.ds(base, tw)])


_combine = functools.partial(
    pl.kernel,
    out_type=jax.ShapeDtypeStruct((T, D), jnp.float32),
    mesh=plsc.VectorSubcoreMesh(core_axis_name="c", subcore_axis_name="s"),
    scratch_types=[
        pltpu.VMEM((T // (NC * NSUB),), jnp.int32),
        pltpu.VMEM((T // (NC * NSUB),), jnp.int32),
        pltpu.VMEM((T // (NC * NSUB), D), jnp.float32),
        pltpu.VMEM((T // (NC * NSUB), D), jnp.float32),
        pltpu.SemaphoreType.DMA,
        pltpu.SemaphoreType.DMA,
    ],
)(_combine_body)


def kernel(hidden, gate_logits, W_experts, b_experts):
    xs, ws, pos, texp = _dispatch(gate_logits.T, hidden)
    y = pl.pallas_call(
        _mm_body,
        grid_spec=pltpu.PrefetchScalarGridSpec(
            num_scalar_prefetch=1,
            grid=(NT,),
            in_specs=[
                pl.BlockSpec((MM_TILE, D), lambda i, tx: (i, 0)),
                pl.BlockSpec((1, D, D), lambda i, tx: (tx[i], 0, 0)),
                pl.BlockSpec((1, 1, D), lambda i, tx: (tx[i], 0, 0)),
                pl.BlockSpec((1, 1, MM_TILE), lambda i, tx: (i, 0, 0)),
            ],
            out_specs=pl.BlockSpec((MM_TILE, D), lambda i, tx: (i, 0)),
        ),
        out_shape=jax.ShapeDtypeStruct((NS, D), jnp.float32),
    )(texp, xs, W_experts, b_experts.reshape(E, 1, D), ws.reshape(NT, 1, MM_TILE))
    return _combine(y, pos)
